# baseline (speedup 1.0000x reference)
import functools

import jax
import jax.numpy as jnp
from jax import lax
from jax.experimental import pallas as pl
from jax.experimental.pallas import tpu as pltpu
from jax.experimental.pallas import tpu_sc as plsc

_ROWS = 4096
_COLS = 8192
_NPROBE = 128
_NC = 2
_NS = 16
_NW = _NC * _NS
_LANES = 16

_TC_ROWS = 1536
_SC_ROWS = _ROWS - _TC_ROWS
_RPW = _SC_ROWS // _NW
_BLK = 128


def _sc_body(x_hbm, probe_hbm, out_hbm, probe_v, idx_v, buf_v, sem):
    wid = lax.axis_index("s") * _NC + lax.axis_index("c")
    base_row = _TC_ROWS + wid * _RPW

    pltpu.sync_copy(probe_hbm, probe_v)

    for m in range(_NPROBE // _LANES):
        sl = pl.ds(m * _LANES, _LANES)
        c = probe_v[sl]
        probe_v[sl] = ((c >> 7) << 10) + (c & 127)

    def build_fire(k, carry):
        i = base_row + k
        off = (i >> 3) * 65536 + (i & 7) * 128
        for m in range(_NPROBE // _LANES):
            sl = pl.ds(m * _LANES, _LANES)
            idx_v[k, sl] = probe_v[sl] + off
        pltpu.async_copy(x_hbm.at[idx_v.at[k]], buf_v.at[k], sem)
        return carry

    lax.fori_loop(0, _RPW, build_fire, 0, unroll=False)

    def drain(k, carry):
        pltpu.make_async_copy(x_hbm.at[idx_v.at[k]], buf_v.at[k], sem).wait()
        return carry

    lax.fori_loop(0, _RPW, drain, 0, unroll=False)

    pltpu.sync_copy(buf_v, out_hbm.at[pl.ds(base_row, _RPW)])


_sc_gather = functools.partial(
    pl.kernel,
    out_type=jax.ShapeDtypeStruct((_ROWS, _NPROBE), jnp.float32),
    mesh=plsc.VectorSubcoreMesh(
        core_axis_name="c", subcore_axis_name="s",
        num_cores=_NC, num_subcores=_NS,
    ),
    scratch_types=[
        pltpu.VMEM((_NPROBE,), jnp.int32),
        pltpu.VMEM((_RPW, _NPROBE), jnp.int32),
        pltpu.VMEM((_RPW, _NPROBE), jnp.float32),
        pltpu.SemaphoreType.DMA,
    ],
)(_sc_body)


def _tc_body(idx_ref, x_ref, o_ref, onehot_ref):
    @pl.when(pl.program_id(0) == 0)
    def _build_onehot():
        idx = idx_ref[0, :]
        cols = jax.lax.broadcasted_iota(jnp.int32, (_COLS, _NPROBE), 0)
        onehot_ref[...] = (cols == idx[None, :]).astype(jnp.bfloat16)

    xb = x_ref[...]
    hi = xb.astype(jnp.bfloat16)
    lo = (xb - hi.astype(jnp.float32)).astype(jnp.bfloat16)
    oh = onehot_ref[...]
    acc = jax.lax.dot_general(
        hi, oh, (((1,), (0,)), ((), ())), preferred_element_type=jnp.float32
    )
    acc += jax.lax.dot_general(
        lo, oh, (((1,), (0,)), ((), ())), preferred_element_type=jnp.float32
    )
    o_ref[...] = acc


def _tc_gather(x, idx2d):
    return pl.pallas_call(
        _tc_body,
        grid=(_TC_ROWS // _BLK,),
        in_specs=[
            pl.BlockSpec((1, _NPROBE), lambda i: (0, 0)),
            pl.BlockSpec((_BLK, _COLS), lambda i: (i, 0)),
        ],
        out_specs=pl.BlockSpec((_BLK, _NPROBE), lambda i: (i, 0)),
        out_shape=jax.ShapeDtypeStruct((_TC_ROWS, _NPROBE), jnp.float32),
        scratch_shapes=[pltpu.VMEM((_COLS, _NPROBE), jnp.bfloat16)],
        compiler_params=pltpu.CompilerParams(
            dimension_semantics=("arbitrary",),
        ),
    )(idx2d, x)


@jax.jit
def kernel(x, probe_idx):
    probe_i32 = probe_idx.astype(jnp.int32)
    xflat = (
        x.reshape(_ROWS // 8, 8, _COLS // 128, 128)
        .transpose(0, 2, 1, 3)
        .reshape(_ROWS * _COLS)
    )
    out_full = _sc_gather(xflat, probe_i32)
    out_tc = _tc_gather(x, probe_i32.reshape(1, _NPROBE))
    return lax.dynamic_update_slice(out_full, out_tc, (0, 0))


# speedup vs baseline: 1.1124x; 1.1124x over previous
import functools

import jax
import jax.numpy as jnp
from jax import lax
from jax.experimental import pallas as pl
from jax.experimental.pallas import tpu as pltpu
from jax.experimental.pallas import tpu_sc as plsc

_ROWS = 4096
_COLS = 8192
_NPROBE = 128
_NC = 2
_NS = 16
_NW = _NC * _NS
_RPW = _ROWS // _NW
_LANES = 16


def _sc_body(x_hbm, probe_hbm, out_hbm, probe_v, idx_v, buf_v, sem):
    wid = lax.axis_index("s") * _NC + lax.axis_index("c")
    base_row = wid * _RPW

    pltpu.sync_copy(probe_hbm, probe_v)

    for m in range(_NPROBE // _LANES):
        sl = pl.ds(m * _LANES, _LANES)
        c = probe_v[sl]
        probe_v[sl] = ((c >> 7) << 10) + (c & 127)

    def build_fire(k, carry):
        i = base_row + k
        off = (i >> 3) * 65536 + (i & 7) * 128
        for m in range(_NPROBE // _LANES):
            sl = pl.ds(m * _LANES, _LANES)
            idx_v[k, sl] = probe_v[sl] + off
        pltpu.async_copy(x_hbm.at[idx_v.at[k]], buf_v.at[k], sem)
        return carry

    lax.fori_loop(0, _RPW, build_fire, 0, unroll=False)

    pltpu.make_async_copy(
        out_hbm.at[pl.ds(base_row, _RPW)], buf_v, sem
    ).wait()

    pltpu.sync_copy(buf_v, out_hbm.at[pl.ds(base_row, _RPW)])


_sc_gather = functools.partial(
    pl.kernel,
    out_type=jax.ShapeDtypeStruct((_ROWS, _NPROBE), jnp.float32),
    mesh=plsc.VectorSubcoreMesh(
        core_axis_name="c", subcore_axis_name="s",
        num_cores=_NC, num_subcores=_NS,
    ),
    scratch_types=[
        pltpu.VMEM((_NPROBE,), jnp.int32),
        pltpu.VMEM((_RPW, _NPROBE), jnp.int32),
        pltpu.VMEM((_RPW, _NPROBE), jnp.float32),
        pltpu.SemaphoreType.DMA,
    ],
)(_sc_body)


@jax.jit
def kernel(x, probe_idx):
    xflat = (
        x.reshape(_ROWS // 8, 8, _COLS // 128, 128)
        .transpose(0, 2, 1, 3)
        .reshape(_ROWS * _COLS)
    )
    return _sc_gather(xflat, probe_idx.astype(jnp.int32))
